# baseline (speedup 1.0000x reference)
import functools

import jax
import jax.numpy as jnp
from jax import lax
from jax.experimental import pallas as pl
from jax.experimental.pallas import tpu as pltpu
from jax.experimental.pallas import tpu_sc as plsc

BC = 4096
HW = 4096
K = 256
L = 16
NW = 32
ROWS_PER_W = BC // NW
R = 8
GROUPS = ROWS_PER_W // R
CHUNKS = HW // L

_mesh = plsc.VectorSubcoreMesh(core_axis_name="c", subcore_axis_name="s")


@functools.partial(
    pl.kernel,
    mesh=_mesh,
    out_type=jax.ShapeDtypeStruct((BC * K,), jnp.float32),
    compiler_params=pltpu.CompilerParams(needs_layout_passes=False),
    scratch_types=[
        pltpu.VMEM((HW,), jnp.int32),
        pltpu.VMEM((R, HW), jnp.float32),
        pltpu.VMEM((R * K,), jnp.float32),
    ],
)
def _sumpool_sc(u_hbm, seg_hbm, out_hbm, seg_v, vals_v, acc_v):
    wid = lax.axis_index("s") * 2 + lax.axis_index("c")
    base_row = wid * ROWS_PER_W
    pltpu.sync_copy(seg_hbm, seg_v)

    def group_body(g, carry):
        pltpu.sync_copy(u_hbm.at[pl.ds(base_row + g * R, R), :], vals_v)

        def zero_body(i, c):
            acc_v[pl.ds(i * L, L)] = jnp.zeros((L,), jnp.float32)
            return c

        lax.fori_loop(0, R * K // L, zero_body, 0)

        def chunk_body(c, cc):
            idx = seg_v[pl.ds(c * L, L)]
            for r in range(R):
                v = vals_v[r, pl.ds(c * L, L)]
                plsc.addupdate_scatter(acc_v, [idx + (r * K)], v)
            return cc

        lax.fori_loop(0, CHUNKS, chunk_body, 0)
        pltpu.sync_copy(acc_v, out_hbm.at[pl.ds(base_row * K + g * R * K, R * K)])
        return carry

    lax.fori_loop(0, GROUPS, group_body, 0)


def kernel(u, segment_ids):
    Bp, Cp, Hp, Wp = u.shape
    u2 = u.reshape(Bp * Cp, Hp * Wp)
    seg32 = segment_ids.astype(jnp.int32)
    out_flat = _sumpool_sc(u2, seg32)
    return out_flat.reshape(Bp, Cp, K)


# speedup vs baseline: 1.2231x; 1.2231x over previous
import functools

import jax
import jax.numpy as jnp
from jax import lax
from jax.experimental import pallas as pl
from jax.experimental.pallas import tpu as pltpu
from jax.experimental.pallas import tpu_sc as plsc

BC = 4096
HW = 4096
K = 256
L = 16
NW = 32
ROWS_PER_W = BC // NW
R = 8
GROUPS = ROWS_PER_W // R
CHUNKS = HW // L

_mesh = plsc.VectorSubcoreMesh(core_axis_name="c", subcore_axis_name="s")


@functools.partial(
    pl.kernel,
    mesh=_mesh,
    out_type=jax.ShapeDtypeStruct((BC * K,), jnp.float32),
    compiler_params=pltpu.CompilerParams(needs_layout_passes=False),
    scratch_types=[
        pltpu.VMEM((HW + L,), jnp.int32),
        pltpu.VMEM((HW,), jnp.int32),
        pltpu.VMEM((R, HW), jnp.float32),
        pltpu.VMEM((R * K,), jnp.float32),
    ],
)
def _sumpool_sc(u_hbm, seg_hbm, out_hbm, seg_v, segb_v, vals_v, acc_v):
    wid = lax.axis_index("s") * 2 + lax.axis_index("c")
    base_row = wid * ROWS_PER_W
    pltpu.sync_copy(seg_hbm, seg_v.at[pl.ds(0, HW)])

    lanes = lax.iota(jnp.int32, L)
    m15 = lanes == (L - 1)

    def shift_body(c, carry):
        sh = plsc.load_gather(seg_v, [lanes + (c * L + 1)])
        segb_v[pl.ds(c * L, L)] = sh
        return carry

    lax.fori_loop(0, CHUNKS, shift_body, 0)

    def group_body(g, carry):
        pltpu.sync_copy(u_hbm.at[pl.ds(base_row + g * R, R), :], vals_v)

        def zero_body(i, c):
            acc_v[pl.ds(i * L, L)] = jnp.zeros((L,), jnp.float32)
            return c

        lax.fori_loop(0, R * K // L, zero_body, 0)

        def chunk_body(c, cc):
            raw_a = seg_v[pl.ds(c * L, L)]
            raw_b = segb_v[pl.ds(c * L, L)]
            neq = raw_a != raw_b
            mask_a = jnp.logical_or(neq, m15)
            mask_b = jnp.logical_and(neq, jnp.logical_not(m15))
            for r in range(R):
                v = vals_v[r, pl.ds(c * L, L)]
                s = plsc.cumsum(v)
                plsc.addupdate_scatter(acc_v, [raw_a + (r * K)], s,
                                       mask=mask_a)
                plsc.addupdate_scatter(acc_v, [raw_b + (r * K)], -s,
                                       mask=mask_b)
            return cc

        lax.fori_loop(0, CHUNKS, chunk_body, 0)
        pltpu.sync_copy(acc_v,
                        out_hbm.at[pl.ds((base_row + g * R) * K, R * K)])
        return carry

    lax.fori_loop(0, GROUPS, group_body, 0)


def kernel(u, segment_ids):
    Bp, Cp, Hp, Wp = u.shape
    u2 = u.reshape(Bp * Cp, Hp * Wp)
    seg32 = segment_ids.astype(jnp.int32)
    out_flat = _sumpool_sc(u2, seg32)
    return out_flat.reshape(Bp, Cp, K)


# speedup vs baseline: 2.2130x; 1.8093x over previous
import functools

import jax
import jax.numpy as jnp
from jax import lax
from jax.experimental import pallas as pl
from jax.experimental.pallas import tpu as pltpu
from jax.experimental.pallas import tpu_sc as plsc

BC = 4096
HW = 4096
K = 256
L = 16
NW = 32
ROWS_PER_W = BC // NW
R = 8
GROUPS = ROWS_PER_W // R
CHUNKS = HW // L

_mesh = plsc.VectorSubcoreMesh(core_axis_name="c", subcore_axis_name="s")


@functools.partial(
    pl.kernel,
    mesh=_mesh,
    out_type=jax.ShapeDtypeStruct((BC * K,), jnp.float32),
    compiler_params=pltpu.CompilerParams(needs_layout_passes=False),
    scratch_types=[
        pltpu.VMEM((HW + L,), jnp.int32),
        pltpu.VMEM((HW,), jnp.int32),
        pltpu.VMEM((R, HW), jnp.float32),
        pltpu.VMEM((R * K,), jnp.float32),
    ],
)
def _sumpool_sc(u_hbm, seg_hbm, out_hbm, seg_v, segb_v, vals_v, acc_v):
    wid = lax.axis_index("s") * 2 + lax.axis_index("c")
    base_row = wid * ROWS_PER_W
    pltpu.sync_copy(seg_hbm, seg_v.at[pl.ds(0, HW)])

    lanes = lax.iota(jnp.int32, L)
    m15 = lanes == (L - 1)

    def shift_body(c, carry):
        sh = plsc.load_gather(seg_v, [lanes + (c * L + 1)])
        segb_v[pl.ds(c * L, L)] = sh
        return carry

    lax.fori_loop(0, CHUNKS, shift_body, 0)

    def group_body(g, carry):
        pltpu.sync_copy(u_hbm.at[pl.ds(base_row + g * R, R), :], vals_v)

        def zero_body(i, c):
            acc_v[pl.ds(i * L, L)] = jnp.zeros((L,), jnp.float32)
            return c

        lax.fori_loop(0, R * K // L, zero_body, 0)

        def chunk_body(c, cc):
            raw_a = seg_v[pl.ds(c * L, L)]
            raw_b = segb_v[pl.ds(c * L, L)]
            neq = raw_a != raw_b
            mask_a = jnp.logical_or(neq, m15)
            mask_b = jnp.logical_and(neq, jnp.logical_not(m15))
            scans = [plsc.cumsum(vals_v[r, pl.ds(c * L, L)])
                     for r in range(R)]
            for r in range(R):
                acc_r = acc_v.at[pl.ds(r * K, K)]
                plsc.addupdate_scatter(acc_r, [raw_a], scans[r],
                                       mask=mask_a)
                plsc.addupdate_scatter(acc_r, [raw_b], -scans[r],
                                       mask=mask_b)
            return cc

        lax.fori_loop(0, CHUNKS, chunk_body, 0)
        pltpu.sync_copy(acc_v,
                        out_hbm.at[pl.ds((base_row + g * R) * K, R * K)])
        return carry

    lax.fori_loop(0, GROUPS, group_body, 0)


def kernel(u, segment_ids):
    Bp, Cp, Hp, Wp = u.shape
    u2 = u.reshape(Bp * Cp, Hp * Wp)
    seg32 = segment_ids.astype(jnp.int32)
    out_flat = _sumpool_sc(u2, seg32)
    return out_flat.reshape(Bp, Cp, K)


# speedup vs baseline: 2.4239x; 1.0953x over previous
import functools

import jax
import jax.numpy as jnp
from jax import lax
from jax.experimental import pallas as pl
from jax.experimental.pallas import tpu as pltpu
from jax.experimental.pallas import tpu_sc as plsc

BC = 4096
HW = 4096
K = 256
L = 16
NW = 32
ROWS_PER_W = BC // NW
R = 8
GROUPS = ROWS_PER_W // R
CHUNKS = HW // L

_mesh = plsc.VectorSubcoreMesh(core_axis_name="c", subcore_axis_name="s")


@functools.partial(
    pl.kernel,
    mesh=_mesh,
    out_type=jax.ShapeDtypeStruct((BC * K,), jnp.float32),
    compiler_params=pltpu.CompilerParams(needs_layout_passes=False),
    scratch_types=[
        pltpu.VMEM((HW + L,), jnp.int32),
        pltpu.VMEM((HW,), jnp.int32),
        pltpu.VMEM((2, R, HW), jnp.float32),
        pltpu.VMEM((ROWS_PER_W * K,), jnp.float32),
        pltpu.SemaphoreType.DMA,
        pltpu.SemaphoreType.DMA,
    ],
)
def _sumpool_sc(u_hbm, seg_hbm, out_hbm, seg_v, segb_v, vals_v, acc_v,
                sem0, sem1):
    wid = lax.axis_index("s") * 2 + lax.axis_index("c")
    base_row = wid * ROWS_PER_W
    pltpu.sync_copy(seg_hbm, seg_v.at[pl.ds(0, HW)])

    lanes = lax.iota(jnp.int32, L)
    m15 = lanes == (L - 1)

    def shift_body(c, carry):
        sh = plsc.load_gather(seg_v, [lanes + (c * L + 1)])
        segb_v[pl.ds(c * L, L)] = sh
        return carry

    lax.fori_loop(0, CHUNKS, shift_body, 0)

    def zero_body(i, c):
        acc_v[pl.ds(i * L, L)] = jnp.zeros((L,), jnp.float32)
        return c

    lax.fori_loop(0, ROWS_PER_W * K // L, zero_body, 0)

    bufs = (vals_v.at[0], vals_v.at[1])
    sems = (sem0, sem1)

    def in_copy(g, par):
        return pltpu.make_async_copy(
            u_hbm.at[pl.ds(base_row + g * R, R), :], bufs[par], sems[par])

    in_copy(0, 0).start()

    def reduce_group(g, par):
        in_copy(g, par).wait()

        def chunk_body(c, cc):
            raw_a = seg_v[pl.ds(c * L, L)]
            raw_b = segb_v[pl.ds(c * L, L)]
            neq = raw_a != raw_b
            mask_a = jnp.logical_or(neq, m15)
            mask_b = jnp.logical_and(neq, jnp.logical_not(m15))
            scans = [plsc.cumsum(bufs[par][r, pl.ds(c * L, L)])
                     for r in range(R)]
            for r in range(R):
                acc_r = acc_v.at[pl.ds((g * R + r) * K, K)]
                plsc.addupdate_scatter(acc_r, [raw_a], scans[r],
                                       mask=mask_a)
                plsc.addupdate_scatter(acc_r, [raw_b], -scans[r],
                                       mask=mask_b)
            return cc

        lax.fori_loop(0, CHUNKS, chunk_body, 0)

    def pair_body(p, carry):
        g0 = 2 * p
        in_copy(g0 + 1, 1).start()
        reduce_group(g0, 0)

        @pl.when(p < GROUPS // 2 - 1)
        def _():
            in_copy(g0 + 2, 0).start()

        reduce_group(g0 + 1, 1)
        return carry

    lax.fori_loop(0, GROUPS // 2, pair_body, 0)
    pltpu.sync_copy(acc_v,
                    out_hbm.at[pl.ds(base_row * K, ROWS_PER_W * K)])


def kernel(u, segment_ids):
    Bp, Cp, Hp, Wp = u.shape
    u2 = u.reshape(Bp * Cp, Hp * Wp)
    seg32 = segment_ids.astype(jnp.int32)
    out_flat = _sumpool_sc(u2, seg32)
    return out_flat.reshape(Bp, Cp, K)
